# Initial kernel scaffold; baseline (speedup 1.0000x reference)
import jax
import jax.numpy as jnp
from jax.experimental import pallas as pl


def kernel(node_s, node_v, edge_s, edge_v, params, edge_index, batch):
    raise NotImplementedError("write your pallas kernel here")


# speedup vs baseline: 9.6981x; 9.6981x over previous
import functools

import jax
import jax.numpy as jnp
from jax.experimental import pallas as pl
from jax.experimental.pallas import tpu as pltpu

EPS = 1e-8
F32 = jnp.float32

NS = 64
NV = 16
PACK = NS + 3 * NV


def _pick_block(n, target):
    b = min(n, target)
    while b > 1:
        if n % b == 0 and (b % 8 == 0 or b == n):
            return b
        b -= 1
    return 1


def _ln(x, g, b):
    mu = jnp.mean(x, axis=-1, keepdims=True)
    xc = x - mu
    var = jnp.mean(xc * xc, axis=-1, keepdims=True)
    return xc / jnp.sqrt(var + 1e-5) * g + b


def _vsq(v3):
    return v3[0] * v3[0] + v3[1] * v3[1] + v3[2] * v3[2]


def _gvp(s, v3, wh, wsw, wsb, wv, relu, gate):
    vh = [jnp.dot(v, wh, preferred_element_type=F32) for v in v3]
    vn = jnp.sqrt(jnp.maximum(_vsq(vh), EPS))
    so = jnp.dot(jnp.concatenate([s, vn], axis=1), wsw,
                 preferred_element_type=F32) + wsb
    if relu:
        so = jnp.maximum(so, 0.0)
    vo = None
    if wv is not None:
        vo = [jnp.dot(h, wv, preferred_element_type=F32) for h in vh]
        if gate:
            g = jax.nn.sigmoid(jnp.sqrt(jnp.maximum(_vsq(vo), EPS)))
            vo = [x * g for x in vo]
    return so, vo


def _ln_gvp(s, v3, g, b):
    s = _ln(s, g, b)
    ch = jnp.maximum(_vsq(v3), EPS)
    den = jnp.sqrt(jnp.mean(ch, axis=1, keepdims=True))
    return s, [x / den for x in v3]


def _split_packed(x, nv):
    s = x[:, :NS]
    v = [x[:, NS + nv * d:NS + nv * (d + 1)] for d in range(3)]
    return s, v


def _full(a):
    return pl.BlockSpec(a.shape, lambda i: (0,) * a.ndim)


def _row_spec(blk, a):
    return pl.BlockSpec((blk,) + a.shape[1:], lambda i: (i,) + (0,) * (a.ndim - 1))


def _call_rowwise(body, data, weights, out_dim, target_blk):
    n = data[0].shape[0]
    blk = _pick_block(n, target_blk)
    return pl.pallas_call(
        body,
        grid=(n // blk,),
        in_specs=[_row_spec(blk, a) for a in data] + [_full(w) for w in weights],
        out_specs=pl.BlockSpec((blk, out_dim), lambda i: (i, 0)),
        out_shape=jax.ShapeDtypeStruct((n, out_dim), F32),
    )(*data, *weights)


def _embed_node_body(s_ref, v9_ref, g, b, wh, wsw, wsb, wv, out_ref):
    s = s_ref[...]
    v9 = v9_ref[...]
    v = [v9[:, 3 * d:3 * d + 3] for d in range(3)]
    s, v = _ln_gvp(s, v, g[...], b[...])
    s, v = _gvp(s, v, wh[...], wsw[...], wsb[...], wv[...], False, False)
    out_ref[...] = jnp.concatenate([s] + v, axis=1)


def _embed_edge_body(s_ref, ev_ref, g, b, wh, wsw, wsb, wv, out_ref):
    s = s_ref[...]
    ev = ev_ref[...]
    v = [ev[:, d:d + 1] for d in range(3)]
    s, v = _ln_gvp(s, v, g[...], b[...])
    s, v = _gvp(s, v, wh[...], wsw[...], wsb[...], wv[...], False, False)
    out_ref[...] = jnp.concatenate([s] + v, axis=1)


def _msg_body(gs_ref, gd_ref, ee_ref,
              wh0, ww0, wb0, wv0, wh1, ww1, wb1, wv1, wh2, ww2, wb2, wv2,
              out_ref):
    gs = gs_ref[...]
    gd = gd_ref[...]
    ee = ee_ref[...]
    ss, vs = _split_packed(gs, NV)
    sd, vd = _split_packed(gd, NV)
    es = ee[:, :32]
    ev = ee[:, 32:]
    ms = jnp.concatenate([ss, es, sd], axis=1)
    mv = [jnp.concatenate([vs[d], ev[:, d:d + 1], vd[d]], axis=1)
          for d in range(3)]
    s, v = _gvp(ms, mv, wh0[...], ww0[...], wb0[...], wv0[...], True, True)
    s, v = _gvp(s, v, wh1[...], ww1[...], wb1[...], wv1[...], True, True)
    s, v = _gvp(s, v, wh2[...], ww2[...], wb2[...], wv2[...], False, False)
    out_ref[...] = jnp.concatenate([s] + v, axis=1)


def _node_body(x_ref, agg_ref, inv_ref,
               g0, b0, f0h, f0w, f0b, f0v, f1h, f1w, f1b, f1v, g1, b1,
               out_ref):
    x = x_ref[...]
    agg = agg_ref[...]
    inv = inv_ref[...]
    xs, xv = _split_packed(x, NV)
    as_, av = _split_packed(agg, NV)
    s = xs + as_ * inv
    v = [a + b * inv for a, b in zip(xv, av)]
    s, v = _ln_gvp(s, v, g0[...], b0[...])
    ds, dv = _gvp(s, v, f0h[...], f0w[...], f0b[...], f0v[...], True, True)
    ds, dv = _gvp(ds, dv, f1h[...], f1w[...], f1b[...], f1v[...], False, False)
    s = s + ds
    v = [a + b for a, b in zip(v, dv)]
    s, v = _ln_gvp(s, v, g1[...], b1[...])
    out_ref[...] = jnp.concatenate([s] + v, axis=1)


def _head_body(x_ref, bm_ref,
               g, b, wh, wsw, wsb, lng, lnb, r0w, r0b, r1w, r1b,
               out_ref, acc_ref):
    i = pl.program_id(0)

    @pl.when(i == 0)
    def _():
        acc_ref[...] = jnp.zeros_like(acc_ref)

    x = x_ref[...]
    s, v = _split_packed(x, NV)
    s, v = _ln_gvp(s, v, g[...], b[...])
    o, _ = _gvp(s, v, wh[...], wsw[...], wsb[...], None, True, False)
    o = _ln(o, lng[...], lnb[...])
    acc_ref[...] += jax.lax.dot_general(
        bm_ref[...], o, (((0,), (0,)), ((), ())), preferred_element_type=F32)

    @pl.when(i == pl.num_programs(0) - 1)
    def _():
        acc = acc_ref[...]
        h = jnp.maximum(
            jnp.dot(acc, r0w[...], preferred_element_type=F32) + r0b[...], 0.0)
        out_ref[...] = jnp.dot(h, r1w[...], preferred_element_type=F32) + r1b[...]


def _head(x, bmask, params):
    n = x.shape[0]
    blk = _pick_block(n, 2000)
    p = params
    w = [p['wout_ln']['g'][None, :], p['wout_ln']['b'][None, :],
         p['wout_gvp']['wh'], p['wout_gvp']['ws']['w'],
         p['wout_gvp']['ws']['b'][None, :],
         p['ln']['g'][None, :], p['ln']['b'][None, :],
         p['r0']['w'], p['r0']['b'][None, :],
         p['r1']['w'], p['r1']['b'][None, :]]
    return pl.pallas_call(
        _head_body,
        grid=(n // blk,),
        in_specs=[_row_spec(blk, x), _row_spec(blk, bmask)]
        + [_full(a) for a in w],
        out_specs=pl.BlockSpec((16, 256), lambda i: (0, 0)),
        out_shape=jax.ShapeDtypeStruct((16, 256), F32),
        scratch_shapes=[pltpu.VMEM((16, NS), F32)],
    )(x, bmask, *w)


def _gvp_weights(p):
    w = [p['wh'], p['ws']['w'], p['ws']['b'][None, :]]
    w.append(p.get('wv'))
    return w


def kernel(node_s, node_v, edge_s, edge_v, params, edge_index, batch):
    n = node_s.shape[0]
    e = edge_s.shape[0]
    src, dst = edge_index[0], edge_index[1]

    v9 = jnp.transpose(node_v, (0, 2, 1)).reshape(n, 9)
    ev3 = edge_v[:, 0, :]

    pn = params['wv_ln']
    x = _call_rowwise(
        _embed_node_body, [node_s, v9],
        [pn['g'][None, :], pn['b'][None, :]] + _gvp_weights(params['wv_gvp']),
        PACK, 2000)

    pe = params['we_ln']
    ee = _call_rowwise(
        _embed_edge_body, [edge_s, ev3],
        [pe['g'][None, :], pe['b'][None, :]] + _gvp_weights(params['we_gvp']),
        35, 3200)

    cnt = jax.ops.segment_sum(jnp.ones((e,), F32), dst, num_segments=n)
    inv = (1.0 / jnp.maximum(cnt, 1.0))[:, None]

    for lp in params['layers']:
        gs = jnp.take(x, src, axis=0)
        gd = jnp.take(x, dst, axis=0)
        c = lp['conv']
        msg = _call_rowwise(
            _msg_body, [gs, gd, ee],
            _gvp_weights(c['m0']) + _gvp_weights(c['m1']) + _gvp_weights(c['m2']),
            PACK, 3200)
        agg = jax.ops.segment_sum(msg, dst, num_segments=n)
        x = _call_rowwise(
            _node_body, [x, agg, inv],
            [lp['norm0']['g'][None, :], lp['norm0']['b'][None, :]]
            + _gvp_weights(lp['f0']) + _gvp_weights(lp['f1'])
            + [lp['norm1']['g'][None, :], lp['norm1']['b'][None, :]],
            PACK, 2000)

    bmask = (batch[:, None] == jnp.arange(16, dtype=batch.dtype)[None, :]).astype(F32)
    return _head(x, bmask, params)


# speedup vs baseline: 14.4895x; 1.4941x over previous
import functools

import jax
import jax.numpy as jnp
from jax import lax
from jax.experimental import pallas as pl
from jax.experimental.pallas import tpu as pltpu
from jax.experimental.pallas import tpu_sc as plsc

EPS = 1e-8
F32 = jnp.float32

NS = 64
NV = 16
PACK = NS + 3 * NV
PACKP = 128


def _pick_block(n, target):
    b = min(n, target)
    while b > 1:
        if n % b == 0 and (b % 8 == 0 or b == n):
            return b
        b -= 1
    return 1


def _ln(x, g, b):
    mu = jnp.mean(x, axis=-1, keepdims=True)
    xc = x - mu
    var = jnp.mean(xc * xc, axis=-1, keepdims=True)
    return xc / jnp.sqrt(var + 1e-5) * g + b


def _vsq(v3):
    return v3[0] * v3[0] + v3[1] * v3[1] + v3[2] * v3[2]


def _gvp(s, v3, wh, wsw, wsb, wv, relu, gate):
    vh = [jnp.dot(v, wh, preferred_element_type=F32) for v in v3]
    vn = jnp.sqrt(jnp.maximum(_vsq(vh), EPS))
    so = jnp.dot(jnp.concatenate([s, vn], axis=1), wsw,
                 preferred_element_type=F32) + wsb
    if relu:
        so = jnp.maximum(so, 0.0)
    vo = None
    if wv is not None:
        vo = [jnp.dot(h, wv, preferred_element_type=F32) for h in vh]
        if gate:
            g = jax.nn.sigmoid(jnp.sqrt(jnp.maximum(_vsq(vo), EPS)))
            vo = [x * g for x in vo]
    return so, vo


def _ln_gvp(s, v3, g, b):
    s = _ln(s, g, b)
    ch = jnp.maximum(_vsq(v3), EPS)
    den = jnp.sqrt(jnp.mean(ch, axis=1, keepdims=True))
    return s, [x / den for x in v3]


def _split_packed(x, nv):
    s = x[:, :NS]
    v = [x[:, NS + nv * d:NS + nv * (d + 1)] for d in range(3)]
    return s, v


def _full(a):
    return pl.BlockSpec(a.shape, lambda i: (0,) * a.ndim)


def _row_spec(blk, a):
    return pl.BlockSpec((blk,) + a.shape[1:], lambda i: (i,) + (0,) * (a.ndim - 1))


def _call_rowwise(body, data, weights, out_dim, target_blk):
    n = data[0].shape[0]
    blk = _pick_block(n, target_blk)
    return pl.pallas_call(
        body,
        grid=(n // blk,),
        in_specs=[_row_spec(blk, a) for a in data] + [_full(w) for w in weights],
        out_specs=pl.BlockSpec((blk, out_dim), lambda i: (i, 0)),
        out_shape=jax.ShapeDtypeStruct((n, out_dim), F32),
    )(*data, *weights)


def _embed_node_body(s_ref, v9_ref, g, b, wh, wsw, wsb, wv, out_ref):
    s = s_ref[...]
    v9 = v9_ref[...]
    v = [v9[:, 3 * d:3 * d + 3] for d in range(3)]
    s, v = _ln_gvp(s, v, g[...], b[...])
    s, v = _gvp(s, v, wh[...], wsw[...], wsb[...], wv[...], False, False)
    pad = jnp.zeros((s.shape[0], PACKP - PACK), F32)
    out_ref[...] = jnp.concatenate([s] + v + [pad], axis=1)


def _embed_edge_body(s_ref, ev_ref, g, b, wh, wsw, wsb, wv, out_ref):
    s = s_ref[...]
    ev = ev_ref[...]
    v = [ev[:, d:d + 1] for d in range(3)]
    s, v = _ln_gvp(s, v, g[...], b[...])
    s, v = _gvp(s, v, wh[...], wsw[...], wsb[...], wv[...], False, False)
    out_ref[...] = jnp.concatenate([s] + v, axis=1)


def _sc_gather(table, idx):
    d = table.shape[1]
    b = idx.shape[0]
    nw = 32
    bpw = b // nw
    chunk = 200
    nchunks = bpw // chunk
    mesh = plsc.VectorSubcoreMesh(core_axis_name="c", subcore_axis_name="s")

    @functools.partial(
        pl.kernel,
        out_type=jax.ShapeDtypeStruct((b, d), F32),
        mesh=mesh,
        scratch_types=[
            pltpu.VMEM((bpw,), jnp.int32),
            pltpu.VMEM((2, chunk, d), F32),
            pltpu.SemaphoreType.DMA,
            pltpu.SemaphoreType.DMA,
        ],
    )
    def k(table_hbm, idx_hbm, out_hbm, idx_v, rows_v, sem_g, sem_o):
        wid = lax.axis_index("s") * 2 + lax.axis_index("c")
        base = wid * bpw
        pltpu.sync_copy(idx_hbm.at[pl.ds(base, bpw)], idx_v)

        def g_copy(i, slot):
            return pltpu.make_async_copy(
                table_hbm.at[idx_v.at[pl.ds(i * chunk, chunk)]],
                rows_v.at[slot], sem_g)

        def o_copy(i, slot):
            return pltpu.make_async_copy(
                rows_v.at[slot], out_hbm.at[pl.ds(base + i * chunk, chunk)],
                sem_o)

        g_copy(0, 0).start()

        def pair(j, carry):
            for s in (0, 1):
                i = j * 2 + s
                g_copy(i, s).wait()

                @pl.when(i > 0)
                def _():
                    o_copy(i - 1, 1 - s).wait()

                @pl.when(i + 1 < nchunks)
                def _():
                    g_copy(i + 1, 1 - s).start()

                o_copy(i, s).start()
            return carry

        lax.fori_loop(0, nchunks // 2, pair, 0)
        o_copy(nchunks - 1, 1).wait()

    return k(table, idx)


def _messages(g, ee, weights):
    e = g.shape[0] // 2
    blk = _pick_block(e, 3200)
    nb = e // blk
    in_specs = [
        pl.BlockSpec((blk, PACKP), lambda i: (i, 0)),
        pl.BlockSpec((blk, PACKP), lambda i: (i + nb, 0)),
        pl.BlockSpec((blk, 35), lambda i: (i, 0)),
    ] + [_full(w) for w in weights]
    return pl.pallas_call(
        _msg_body,
        grid=(nb,),
        in_specs=in_specs,
        out_specs=pl.BlockSpec((blk, PACK), lambda i: (i, 0)),
        out_shape=jax.ShapeDtypeStruct((e, PACK), F32),
    )(g, g, ee, *weights)


def _msg_body(gs_ref, gd_ref, ee_ref,
              wh0, ww0, wb0, wv0, wh1, ww1, wb1, wv1, wh2, ww2, wb2, wv2,
              out_ref):
    gs = gs_ref[...]
    gd = gd_ref[...]
    ee = ee_ref[...]
    ss, vs = _split_packed(gs, NV)
    sd, vd = _split_packed(gd, NV)
    es = ee[:, :32]
    ev = ee[:, 32:]
    ms = jnp.concatenate([ss, es, sd], axis=1)
    mv = [jnp.concatenate([vs[d], ev[:, d:d + 1], vd[d]], axis=1)
          for d in range(3)]
    s, v = _gvp(ms, mv, wh0[...], ww0[...], wb0[...], wv0[...], True, True)
    s, v = _gvp(s, v, wh1[...], ww1[...], wb1[...], wv1[...], True, True)
    s, v = _gvp(s, v, wh2[...], ww2[...], wb2[...], wv2[...], False, False)
    out_ref[...] = jnp.concatenate([s] + v, axis=1)


def _node_body(x_ref, agg_ref, inv_ref,
               g0, b0, f0h, f0w, f0b, f0v, f1h, f1w, f1b, f1v, g1, b1,
               out_ref):
    x = x_ref[...]
    agg = agg_ref[...]
    inv = inv_ref[...]
    xs, xv = _split_packed(x, NV)
    as_, av = _split_packed(agg, NV)
    s = xs + as_ * inv
    v = [a + b * inv for a, b in zip(xv, av)]
    s, v = _ln_gvp(s, v, g0[...], b0[...])
    ds, dv = _gvp(s, v, f0h[...], f0w[...], f0b[...], f0v[...], True, True)
    ds, dv = _gvp(ds, dv, f1h[...], f1w[...], f1b[...], f1v[...], False, False)
    s = s + ds
    v = [a + b for a, b in zip(v, dv)]
    s, v = _ln_gvp(s, v, g1[...], b1[...])
    pad = jnp.zeros((s.shape[0], PACKP - PACK), F32)
    out_ref[...] = jnp.concatenate([s] + v + [pad], axis=1)


def _head_body(x_ref, bm_ref,
               g, b, wh, wsw, wsb, lng, lnb, r0w, r0b, r1w, r1b,
               out_ref, acc_ref):
    i = pl.program_id(0)

    @pl.when(i == 0)
    def _():
        acc_ref[...] = jnp.zeros_like(acc_ref)

    x = x_ref[...]
    s, v = _split_packed(x, NV)
    s, v = _ln_gvp(s, v, g[...], b[...])
    o, _ = _gvp(s, v, wh[...], wsw[...], wsb[...], None, True, False)
    o = _ln(o, lng[...], lnb[...])
    acc_ref[...] += jax.lax.dot_general(
        bm_ref[...], o, (((0,), (0,)), ((), ())), preferred_element_type=F32)

    @pl.when(i == pl.num_programs(0) - 1)
    def _():
        acc = acc_ref[...]
        h = jnp.maximum(
            jnp.dot(acc, r0w[...], preferred_element_type=F32) + r0b[...], 0.0)
        out_ref[...] = jnp.dot(h, r1w[...], preferred_element_type=F32) + r1b[...]


def _head(x, bmask, params):
    n = x.shape[0]
    blk = _pick_block(n, 2000)
    p = params
    w = [p['wout_ln']['g'][None, :], p['wout_ln']['b'][None, :],
         p['wout_gvp']['wh'], p['wout_gvp']['ws']['w'],
         p['wout_gvp']['ws']['b'][None, :],
         p['ln']['g'][None, :], p['ln']['b'][None, :],
         p['r0']['w'], p['r0']['b'][None, :],
         p['r1']['w'], p['r1']['b'][None, :]]
    return pl.pallas_call(
        _head_body,
        grid=(n // blk,),
        in_specs=[_row_spec(blk, x), _row_spec(blk, bmask)]
        + [_full(a) for a in w],
        out_specs=pl.BlockSpec((16, 256), lambda i: (0, 0)),
        out_shape=jax.ShapeDtypeStruct((16, 256), F32),
        scratch_shapes=[pltpu.VMEM((16, NS), F32)],
    )(x, bmask, *w)


def _gvp_weights(p):
    w = [p['wh'], p['ws']['w'], p['ws']['b'][None, :]]
    w.append(p.get('wv'))
    return w


def kernel(node_s, node_v, edge_s, edge_v, params, edge_index, batch):
    n = node_s.shape[0]
    e = edge_s.shape[0]
    src, dst = edge_index[0], edge_index[1]

    v9 = jnp.transpose(node_v, (0, 2, 1)).reshape(n, 9)
    ev3 = edge_v[:, 0, :]

    pn = params['wv_ln']
    x = _call_rowwise(
        _embed_node_body, [node_s, v9],
        [pn['g'][None, :], pn['b'][None, :]] + _gvp_weights(params['wv_gvp']),
        PACKP, 2000)

    pe = params['we_ln']
    ee = _call_rowwise(
        _embed_edge_body, [edge_s, ev3],
        [pe['g'][None, :], pe['b'][None, :]] + _gvp_weights(params['we_gvp']),
        35, 3200)

    cnt = jax.ops.segment_sum(jnp.ones((e,), F32), dst, num_segments=n)
    inv = (1.0 / jnp.maximum(cnt, 1.0))[:, None]

    sd_idx = jnp.concatenate([src, dst])
    for lp in params['layers']:
        g = _sc_gather(x, sd_idx)
        c = lp['conv']
        msg = _messages(
            g, ee,
            _gvp_weights(c['m0']) + _gvp_weights(c['m1']) + _gvp_weights(c['m2']))
        agg = jax.ops.segment_sum(msg, dst, num_segments=n)
        x = _call_rowwise(
            _node_body, [x, agg, inv],
            [lp['norm0']['g'][None, :], lp['norm0']['b'][None, :]]
            + _gvp_weights(lp['f0']) + _gvp_weights(lp['f1'])
            + [lp['norm1']['g'][None, :], lp['norm1']['b'][None, :]],
            PACKP, 2000)

    bmask = (batch[:, None] == jnp.arange(16, dtype=batch.dtype)[None, :]).astype(F32)
    return _head(x, bmask, params)
